# baseline (speedup 1.0000x reference)
import functools

import jax
import jax.numpy as jnp
from jax import lax
from jax.experimental import pallas as pl
from jax.experimental.pallas import tpu as pltpu
from jax.experimental.pallas import tpu_sc as plsc

NUM_ITEM = 100000
EMBED_DIM = 128
BATCH = 16384
N_COLS = 10

_info = plsc.get_sparse_core_info()
_NC, _NS = _info.num_cores, _info.num_subcores
_NW = _NC * _NS
_CHUNK = 128
_B_PER_W = BATCH // _NW
_NCH = _B_PER_W // _CHUNK
_L = 16

_mesh = plsc.VectorSubcoreMesh(core_axis_name="c", subcore_axis_name="s")


@functools.partial(
    pl.kernel,
    mesh=_mesh,
    out_type=jax.ShapeDtypeStruct((BATCH, EMBED_DIM), jnp.float32),
    scratch_types=[
        pltpu.VMEM((_NCH, _CHUNK), jnp.int32),
        pltpu.VMEM((_NCH, _CHUNK), jnp.int32),
        pltpu.VMEM((_NCH, _CHUNK, EMBED_DIM), jnp.float32),
    ]
    + [pltpu.SemaphoreType.DMA] * (3 * _NCH),
)
def _gather_kernel(fea_hbm, table_hbm, out_hbm, pos_v, idx_v, rows_v, *sems):
    psems = sems[: _NCH]
    gsems = sems[_NCH : 2 * _NCH]
    wsems = sems[2 * _NCH :]
    wid = lax.axis_index("s") * _NC + lax.axis_index("c")
    rbase = wid * _B_PER_W
    lane = lax.iota(jnp.int32, _L)
    fea_gathers = []
    for j in range(_NCH):
        row = pos_v.at[j]
        for k in range(_CHUNK // _L):
            row[pl.ds(k * _L, _L)] = (
                lane + (rbase + j * _CHUNK + k * _L)
            ) * N_COLS
        fea_gathers.append(
            pltpu.async_copy(fea_hbm.at[row], idx_v.at[j], psems[j])
        )
    gathers = []
    for j in range(_NCH):
        fea_gathers[j].wait()
        gathers.append(
            pltpu.async_copy(table_hbm.at[idx_v.at[j]], rows_v.at[j], gsems[j])
        )
    writes = []
    for j in range(_NCH):
        gathers[j].wait()
        writes.append(
            pltpu.async_copy(
                rows_v.at[j],
                out_hbm.at[pl.ds(rbase + j * _CHUNK, _CHUNK)],
                wsems[j],
            )
        )
    for w in writes:
        w.wait()


def kernel(item_fea, embedding_itemId):
    fea_flat = item_fea.astype(jnp.int32).reshape(BATCH * N_COLS)
    return _gather_kernel(fea_flat, embedding_itemId)


# speedup vs baseline: 1.4804x; 1.4804x over previous
import functools

import jax
import jax.numpy as jnp
from jax import lax
from jax.experimental import pallas as pl
from jax.experimental.pallas import tpu as pltpu
from jax.experimental.pallas import tpu_sc as plsc

NUM_ITEM = 100000
EMBED_DIM = 128
BATCH = 16384

_info = plsc.get_sparse_core_info()
_NC, _NS = _info.num_cores, _info.num_subcores
_NW = _NC * _NS
_CHUNK = 64
_B_PER_W = BATCH // _NW
_NCH = _B_PER_W // _CHUNK

_mesh = plsc.VectorSubcoreMesh(core_axis_name="c", subcore_axis_name="s")


@functools.partial(
    pl.kernel,
    mesh=_mesh,
    out_type=jax.ShapeDtypeStruct((BATCH, EMBED_DIM), jnp.float32),
    scratch_types=[
        pltpu.VMEM((_NCH, _CHUNK), jnp.int32),
        pltpu.VMEM((_NCH, _CHUNK, EMBED_DIM), jnp.float32),
    ]
    + [pltpu.SemaphoreType.DMA] * (2 * _NCH),
)
def _gather_kernel(idx_hbm, table_hbm, out_hbm, idx_v, rows_v, *sems):
    gsems, wsems = sems[:_NCH], sems[_NCH:]
    wid = lax.axis_index("s") * _NC + lax.axis_index("c")
    base = wid * _NCH
    pltpu.sync_copy(idx_hbm.at[pl.ds(base, _NCH)], idx_v)
    gathers = [
        pltpu.async_copy(table_hbm.at[idx_v.at[j]], rows_v.at[j], gsems[j])
        for j in range(_NCH)
    ]
    writes = []
    for j in range(_NCH):
        gathers[j].wait()
        writes.append(
            pltpu.async_copy(
                rows_v.at[j],
                out_hbm.at[pl.ds((base + j) * _CHUNK, _CHUNK)],
                wsems[j],
            )
        )
    for w in writes:
        w.wait()


def kernel(item_fea, embedding_itemId):
    idx = item_fea[:, 0].astype(jnp.int32).reshape(BATCH // _CHUNK, _CHUNK)
    return _gather_kernel(idx, embedding_itemId)
